# Initial kernel scaffold; baseline (speedup 1.0000x reference)
import jax
import jax.numpy as jnp
from jax.experimental import pallas as pl


def kernel(X):
    raise NotImplementedError("write your pallas kernel here")


# speedup vs baseline: 27.2752x; 27.2752x over previous
import jax
import jax.numpy as jnp
from jax.experimental import pallas as pl
from jax.experimental.pallas import tpu as pltpu

_ROWS_PER_BLOCK = 256
_BISECT_ITERS = 16
_POLISH_ITERS = 2


def _sparsemax_block(x_ref, o_ref):
    z = -x_ref[...]
    m = jnp.max(z, axis=-1, keepdims=True)
    lo = m - 1.0
    hi = m
    for _ in range(_BISECT_ITERS):
        mid = 0.5 * (lo + hi)
        f = jnp.sum(jnp.maximum(z - mid, 0.0), axis=-1, keepdims=True)
        ge = f >= 1.0
        lo = jnp.where(ge, mid, lo)
        hi = jnp.where(ge, hi, mid)
    tau = lo
    for _ in range(_POLISH_ITERS):
        mask = z > tau
        k = jnp.sum(mask.astype(jnp.float32), axis=-1, keepdims=True)
        s = jnp.sum(jnp.where(mask, z, 0.0), axis=-1, keepdims=True)
        tau = (s - 1.0) / jnp.maximum(k, 1.0)
    o_ref[...] = jnp.maximum(z - tau, 0.0)


def kernel(X):
    b, t, n = X.shape
    rows = b * t
    x2 = X.reshape(rows, n)
    grid = (rows // _ROWS_PER_BLOCK,)
    out = pl.pallas_call(
        _sparsemax_block,
        grid=grid,
        in_specs=[pl.BlockSpec((_ROWS_PER_BLOCK, n), lambda i: (i, 0))],
        out_specs=pl.BlockSpec((_ROWS_PER_BLOCK, n), lambda i: (i, 0)),
        out_shape=jax.ShapeDtypeStruct((rows, n), X.dtype),
        compiler_params=pltpu.CompilerParams(
            dimension_semantics=("parallel",),
        ),
    )(x2)
    return out.reshape(b, t, n)


# speedup vs baseline: 37.6567x; 1.3806x over previous
import jax
import jax.numpy as jnp
from jax.experimental import pallas as pl
from jax.experimental.pallas import tpu as pltpu

_ROWS_PER_BLOCK = 512
_BISECT_ITERS = 8
_POLISH_ITERS = 3


def _sparsemax_block(x_ref, o_ref):
    z = -x_ref[...]
    m = jnp.max(z, axis=-1, keepdims=True)
    lo = m - 1.0
    hi = m
    for _ in range(_BISECT_ITERS):
        mid = 0.5 * (lo + hi)
        f = jnp.sum(jnp.maximum(z - mid, 0.0), axis=-1, keepdims=True)
        ge = f >= 1.0
        lo = jnp.where(ge, mid, lo)
        hi = jnp.where(ge, hi, mid)
    tau = lo
    for _ in range(_POLISH_ITERS):
        mask = z > tau
        k = jnp.sum(mask.astype(jnp.float32), axis=-1, keepdims=True)
        s = jnp.sum(jnp.where(mask, z, 0.0), axis=-1, keepdims=True)
        tau = (s - 1.0) / jnp.maximum(k, 1.0)
    o_ref[...] = jnp.maximum(z - tau, 0.0)


def kernel(X):
    b, t, n = X.shape
    rows = b * t
    x2 = X.reshape(rows, n)
    grid = (rows // _ROWS_PER_BLOCK,)
    out = pl.pallas_call(
        _sparsemax_block,
        grid=grid,
        in_specs=[pl.BlockSpec((_ROWS_PER_BLOCK, n), lambda i: (i, 0))],
        out_specs=pl.BlockSpec((_ROWS_PER_BLOCK, n), lambda i: (i, 0)),
        out_shape=jax.ShapeDtypeStruct((rows, n), X.dtype),
        compiler_params=pltpu.CompilerParams(
            dimension_semantics=("parallel",),
        ),
    )(x2)
    return out.reshape(b, t, n)


# speedup vs baseline: 50.7081x; 1.3466x over previous
import jax
import jax.numpy as jnp
from jax.experimental import pallas as pl
from jax.experimental.pallas import tpu as pltpu

_ROWS_PER_BLOCK = 512
_BISECT_ITERS = 4
_POLISH_ITERS = 3


def _sparsemax_block(x_ref, o_ref):
    z = -x_ref[...]
    m = jnp.max(z, axis=-1, keepdims=True)
    lo = m - 1.0
    hi = m
    for _ in range(_BISECT_ITERS):
        mid = 0.5 * (lo + hi)
        f = jnp.sum(jnp.maximum(z - mid, 0.0), axis=-1, keepdims=True)
        ge = f >= 1.0
        lo = jnp.where(ge, mid, lo)
        hi = jnp.where(ge, hi, mid)
    tau = lo
    for _ in range(_POLISH_ITERS):
        mask = z > tau
        k = jnp.sum(mask.astype(jnp.float32), axis=-1, keepdims=True)
        s = jnp.sum(jnp.where(mask, z, 0.0), axis=-1, keepdims=True)
        tau = (s - 1.0) / jnp.maximum(k, 1.0)
    o_ref[...] = jnp.maximum(z - tau, 0.0)


def kernel(X):
    b, t, n = X.shape
    rows = b * t
    x2 = X.reshape(rows, n)
    grid = (rows // _ROWS_PER_BLOCK,)
    out = pl.pallas_call(
        _sparsemax_block,
        grid=grid,
        in_specs=[pl.BlockSpec((_ROWS_PER_BLOCK, n), lambda i: (i, 0))],
        out_specs=pl.BlockSpec((_ROWS_PER_BLOCK, n), lambda i: (i, 0)),
        out_shape=jax.ShapeDtypeStruct((rows, n), X.dtype),
        compiler_params=pltpu.CompilerParams(
            dimension_semantics=("parallel",),
        ),
    )(x2)
    return out.reshape(b, t, n)
